# baseline (speedup 1.0000x reference)
import functools

import jax
import jax.numpy as jnp
from jax import lax
from jax.experimental import pallas as pl
from jax.experimental.pallas import tpu as pltpu
from jax.experimental.pallas import tpu_sc as plsc

B_SZ = 1024
SEQ = 512
D = 128
T = B_SZ * SEQ
NC, NS, L = 2, 16, 16
NW = NC * NS
TPW = T // NW
C = 128
N_CHUNKS = TPW // C


def _body(seq_hbm, pos_hbm, seg_hbm, tok_tab, pos_tab, seg_tab, out_hbm,
          idx_t, idx_p, idx_s, buf_t, buf_p, buf_s, sem_t, sem_p, sem_s):
    wid = lax.axis_index("s") * NC + lax.axis_index("c")
    w_base = pl.multiple_of(wid * TPW, C)

    def chunk(c, _):
        base = pl.multiple_of(w_base + c * C, C)
        pltpu.sync_copy(seq_hbm.at[pl.ds(base, C)], idx_t)
        pltpu.sync_copy(pos_hbm.at[pl.ds(base, C)], idx_p)
        pltpu.sync_copy(seg_hbm.at[pl.ds(base, C)], idx_s)
        cp_t = pltpu.async_copy(tok_tab.at[idx_t], buf_t, sem_t)
        cp_p = pltpu.async_copy(pos_tab.at[idx_p], buf_p, sem_p)
        cp_s = pltpu.async_copy(seg_tab.at[idx_s], buf_s, sem_s)
        cp_t.wait()
        cp_p.wait()
        cp_s.wait()

        def row(r, _):
            for g in range(D // L):
                sl = pl.ds(g * L, L)
                buf_t[r, sl] = buf_t[r, sl] + buf_p[r, sl] + buf_s[r, sl]
            return 0

        lax.fori_loop(0, C, row, 0, unroll=False)
        pltpu.sync_copy(buf_t, out_hbm.at[pl.ds(base, C)])
        return 0

    lax.fori_loop(0, N_CHUNKS, chunk, 0, unroll=False)


@jax.jit
def _embed(seq, pos, seg, tok_tab, pos_tab, seg_tab):
    mesh = plsc.VectorSubcoreMesh(core_axis_name="c", subcore_axis_name="s",
                                  num_cores=NC, num_subcores=NS)
    k = pl.kernel(
        _body,
        out_type=jax.ShapeDtypeStruct((T, D), jnp.float32),
        mesh=mesh,
        scratch_types=[
            pltpu.VMEM((C,), jnp.int32),
            pltpu.VMEM((C,), jnp.int32),
            pltpu.VMEM((C,), jnp.int32),
            pltpu.VMEM((C, D), jnp.float32),
            pltpu.VMEM((C, D), jnp.float32),
            pltpu.VMEM((C, D), jnp.float32),
            pltpu.SemaphoreType.DMA,
            pltpu.SemaphoreType.DMA,
            pltpu.SemaphoreType.DMA,
        ],
    )
    return k(seq, pos, seg, tok_tab, pos_tab, seg_tab)


def kernel(sequence, postion_label, segment_label, token_table,
           position_table, segment_table):
    seq = sequence.reshape(T).astype(jnp.int32)
    pos = postion_label.reshape(T).astype(jnp.int32)
    seg = segment_label.reshape(T).astype(jnp.int32)
    out = _embed(seq, pos, seg, token_table, position_table, segment_table)
    return out.reshape(B_SZ, SEQ, D)
